# baseline (speedup 1.0000x reference)
import jax
import jax.numpy as jnp
from jax.experimental import pallas as pl
from jax.experimental.pallas import tpu as pltpu

N = 10000
D = 128
CP = 10240
CH = 128
NCH = CP // CH
NFULL = N // CH
NREM = N - NFULL * CH
K = 32
ALPHA = 3.0
R = 200
ONE_BITS = 0x3F800000


def _nodevec_body(e0_ref, e1_ref, w0t_ref, b0_ref, w1t_ref, b1_ref,
                  nv1_ref, nv2_ref):
    a0 = jax.lax.dot_general(e0_ref[...].astype(jnp.bfloat16),
                             w0t_ref[...].astype(jnp.bfloat16),
                             (((1,), (0,)), ((), ())),
                             preferred_element_type=jnp.float32)
    a1 = jax.lax.dot_general(e1_ref[...].astype(jnp.bfloat16),
                             w1t_ref[...].astype(jnp.bfloat16),
                             (((1,), (0,)), ((), ())),
                             preferred_element_type=jnp.float32)
    nv1_ref[...] = jnp.tanh(ALPHA * (a0 + b0_ref[...]))
    nv2_ref[...] = jnp.tanh(ALPHA * (a1 + b1_ref[...]))


def _adj_body(nv1_ref, nv2_ref, nv2t_ref, nv1t_ref, u_ref, s_ref, b_ref,
              out_ref):
    a = (jax.lax.dot_general(nv1_ref[...], nv2t_ref[...],
                             (((1,), (0,)), ((), ())),
                             preferred_element_type=jnp.float32)
         - jax.lax.dot_general(nv2_ref[...], nv1t_ref[...],
                               (((1,), (0,)), ((), ())),
                               preferred_element_type=jnp.float32))
    def xstep(_, lohi):
        lo, hi = lohi
        mid = lo + (hi - lo) // 2
        x = jax.lax.bitcast_convert_type(mid, jnp.float32)
        sat = jnp.tanh(ALPHA * x) >= 1.0
        return jnp.where(sat, lo, mid), jnp.where(sat, mid, hi)

    xlo0 = jnp.full((1, 1), 0x40000000, jnp.int32)
    xhi0 = jnp.full((1, 1), 0x41000000, jnp.int32)
    _, xhi = jax.lax.fori_loop(0, 26, xstep, (xlo0, xhi0))
    a_c = jax.lax.bitcast_convert_type(xhi, jnp.float32)

    ones = a >= a_c
    onesb = ones.astype(jnp.bfloat16)

    tot = jax.lax.dot_general(onesb, b_ref[...], (((1,), (0,)), ((), ())),
                              preferred_element_type=jnp.float32)
    offs = jax.lax.dot_general(tot.astype(jnp.bfloat16), s_ref[...],
                               (((1,), (0,)), ((), ())),
                               preferred_element_type=jnp.float32)
    n_tot = offs[:, NCH - 1:NCH] + tot[:, NCH - 1:NCH]
    fast = jnp.min(n_tot) >= K

    @pl.when(fast)
    def _():
        u = u_ref[...]
        colmin = jnp.min(offs, axis=0, keepdims=True)
        for j in range(NFULL + 1):
            ncols = CH if j < NFULL else NREM
            out_ref[:, j * CH:j * CH + ncols] = jnp.zeros((R, ncols),
                                                          jnp.float32)
        for j in range(NFULL + 1):
            @pl.when(colmin[0, j] < K)
            def _(j=j):
                pre = jax.lax.dot_general(
                    onesb[:, j * CH:(j + 1) * CH], u, (((1,), (0,)), ((), ())),
                    preferred_element_type=jnp.float32)
                gp = pre + offs[:, j:j + 1]
                keep = ones[:, j * CH:(j + 1) * CH] & (gp <= K)
                ncols = CH if j < NFULL else NREM
                out_ref[:, j * CH:j * CH + ncols] = (
                    keep.astype(jnp.float32)[:, :ncols])

    @pl.when(jnp.logical_not(fast))
    def _():
        v = jnp.maximum(jnp.tanh(ALPHA * a), 0.0)
        bits = jax.lax.bitcast_convert_type(v, jnp.int32)

        def step(_, lohi):
            lo, hi = lohi
            mid = lo + (hi - lo) // 2
            cnt = jnp.sum((bits >= mid).astype(jnp.int32), axis=1,
                          keepdims=True)
            ge = cnt >= K
            return jnp.where(ge, mid, lo), jnp.where(ge, hi, mid)

        lo0 = jnp.zeros((R, 1), jnp.int32)
        hi0 = jnp.full((R, 1), ONE_BITS + 1, jnp.int32)
        lo, _ = jax.lax.fori_loop(0, 31, step, (lo0, hi0))
        t = jax.lax.bitcast_convert_type(lo, jnp.float32)

        c_gt = jnp.sum((v > t).astype(jnp.int32), axis=1, keepdims=True)
        m = K - c_gt
        eq = v == t
        col1 = jax.lax.broadcasted_iota(jnp.int32, (R, CP), 1) + 1

        def istep(_, lohi):
            lo, hi = lohi
            mid = (lo + hi) // 2
            cnt = jnp.sum((eq & (col1 <= mid)).astype(jnp.int32), axis=1,
                          keepdims=True)
            ge = cnt >= m
            return jnp.where(ge, lo, mid), jnp.where(ge, mid, hi)

        ilo0 = jnp.zeros((R, 1), jnp.int32)
        ihi0 = jnp.full((R, 1), CP, jnp.int32)
        _, ihi = jax.lax.fori_loop(0, 14, istep, (ilo0, ihi0))

        mask = (v > t) | (eq & (col1 <= ihi))
        out_ref[...] = (v * mask.astype(jnp.float32))[:, :N]


def _nodevecs(emb0, emb1, W0, b0, W1, b1):
    bs = 1000
    return pl.pallas_call(
        _nodevec_body,
        grid=(N // bs,),
        in_specs=[
            pl.BlockSpec((bs, D), lambda i: (i, 0)),
            pl.BlockSpec((bs, D), lambda i: (i, 0)),
            pl.BlockSpec((D, D), lambda i: (0, 0)),
            pl.BlockSpec((1, D), lambda i: (0, 0)),
            pl.BlockSpec((D, D), lambda i: (0, 0)),
            pl.BlockSpec((1, D), lambda i: (0, 0)),
        ],
        out_specs=[
            pl.BlockSpec((bs, D), lambda i: (i, 0)),
            pl.BlockSpec((bs, D), lambda i: (i, 0)),
        ],
        out_shape=[
            jax.ShapeDtypeStruct((N, D), jnp.float32),
            jax.ShapeDtypeStruct((N, D), jnp.float32),
        ],
    )(emb0, emb1, W0.T, b0.reshape(1, D), W1.T, b1.reshape(1, D))


def _masked_adj(nv1, nv2, nv2t, nv1t, U, S, B):
    return pl.pallas_call(
        _adj_body,
        grid=(N // R,),
        in_specs=[
            pl.BlockSpec((R, D), lambda i: (i, 0)),
            pl.BlockSpec((R, D), lambda i: (i, 0)),
            pl.BlockSpec((D, CP), lambda i: (0, 0)),
            pl.BlockSpec((D, CP), lambda i: (0, 0)),
            pl.BlockSpec((CH, CH), lambda i: (0, 0)),
            pl.BlockSpec((NCH, NCH), lambda i: (0, 0)),
            pl.BlockSpec((CP, NCH), lambda i: (0, 0)),
        ],
        out_specs=pl.BlockSpec((R, N), lambda i: (i, 0)),
        out_shape=jax.ShapeDtypeStruct((N, N), jnp.float32),
    )(nv1, nv2, nv2t, nv1t, U, S, B)


def kernel(emb0, emb1, W0, b0, W1, b1, k):
    nv1, nv2 = _nodevecs(emb0, emb1, W0, b0, W1, b1)
    nv1b = nv1.astype(jnp.bfloat16)
    nv2b = nv2.astype(jnp.bfloat16)
    nv2t = jnp.pad(nv2b.T, ((0, 0), (0, CP - N)))
    nv1t = jnp.pad(nv1b.T, ((0, 0), (0, CP - N)))
    U = jnp.triu(jnp.ones((CH, CH), jnp.bfloat16))
    S = jnp.triu(jnp.ones((NCH, NCH), jnp.bfloat16), k=1)
    B = jnp.repeat(jnp.eye(NCH, dtype=jnp.bfloat16), CH, axis=0)
    return _masked_adj(nv1b, nv2b, nv2t, nv1t, U, S, B)


# speedup vs baseline: 1.8210x; 1.8210x over previous
import jax
import jax.numpy as jnp
from jax.experimental import pallas as pl
from jax.experimental.pallas import tpu as pltpu

N = 10000
D = 128
CP = 10240
CH = 128
NCH = CP // CH
NFULL = N // CH
NREM = N - NFULL * CH
K = 32
ALPHA = 3.0
R = 200
ONE_BITS = 0x3F800000


def _nodevec_body(e0_ref, e1_ref, w0t_ref, b0_ref, w1t_ref, b1_ref,
                  nv1_ref, nv2_ref):
    a0 = jax.lax.dot_general(e0_ref[...].astype(jnp.bfloat16),
                             w0t_ref[...].astype(jnp.bfloat16),
                             (((1,), (0,)), ((), ())),
                             preferred_element_type=jnp.float32)
    a1 = jax.lax.dot_general(e1_ref[...].astype(jnp.bfloat16),
                             w1t_ref[...].astype(jnp.bfloat16),
                             (((1,), (0,)), ((), ())),
                             preferred_element_type=jnp.float32)
    nv1_ref[...] = jnp.tanh(ALPHA * (a0 + b0_ref[...]))
    nv2_ref[...] = jnp.tanh(ALPHA * (a1 + b1_ref[...]))


def _adj_body(h_ref, gt_ref, u_ref, s_ref, out_ref):
    a = jax.lax.dot_general(h_ref[...], gt_ref[...], (((1,), (0,)), ((), ())),
                            preferred_element_type=jnp.float32)
    def xstep(_, lohi):
        lo, hi = lohi
        mid = lo + (hi - lo) // 2
        x = jax.lax.bitcast_convert_type(mid, jnp.float32)
        sat = jnp.tanh(ALPHA * x) >= 1.0
        return jnp.where(sat, lo, mid), jnp.where(sat, mid, hi)

    xlo0 = jnp.full((1, 1), 0x40000000, jnp.int32)
    xhi0 = jnp.full((1, 1), 0x41000000, jnp.int32)
    _, xhi = jax.lax.fori_loop(0, 26, xstep, (xlo0, xhi0))
    a_c = jax.lax.bitcast_convert_type(xhi, jnp.float32)

    ones = a >= a_c

    u = u_ref[...]
    pres = []
    tots = []
    for j in range(NCH):
        eqb = ones[:, j * CH:(j + 1) * CH].astype(jnp.bfloat16)
        pre = jax.lax.dot_general(eqb, u, (((1,), (0,)), ((), ())),
                                  preferred_element_type=jnp.float32)
        pres.append(pre)
        tots.append(pre[:, CH - 1:CH])
    tot = jnp.concatenate(tots, axis=1)
    offs = jax.lax.dot_general(tot.astype(jnp.bfloat16), s_ref[...],
                               (((1,), (0,)), ((), ())),
                               preferred_element_type=jnp.float32)
    n_tot = offs[:, NCH - 1:NCH] + tot[:, NCH - 1:NCH]
    fast = jnp.min(n_tot) >= K

    @pl.when(fast)
    def _():
        for j in range(NFULL + 1):
            gp = pres[j] + offs[:, j:j + 1]
            keep = ones[:, j * CH:(j + 1) * CH] & (gp <= K)
            outj = keep.astype(jnp.float32)
            if j < NFULL:
                out_ref[:, j * CH:(j + 1) * CH] = outj
            else:
                out_ref[:, j * CH:j * CH + NREM] = outj[:, :NREM]

    @pl.when(jnp.logical_not(fast))
    def _():
        v = jnp.maximum(jnp.tanh(ALPHA * a), 0.0)
        bits = jax.lax.bitcast_convert_type(v, jnp.int32)

        def step(_, lohi):
            lo, hi = lohi
            mid = lo + (hi - lo) // 2
            cnt = jnp.sum((bits >= mid).astype(jnp.int32), axis=1,
                          keepdims=True)
            ge = cnt >= K
            return jnp.where(ge, mid, lo), jnp.where(ge, hi, mid)

        lo0 = jnp.zeros((R, 1), jnp.int32)
        hi0 = jnp.full((R, 1), ONE_BITS + 1, jnp.int32)
        lo, _ = jax.lax.fori_loop(0, 31, step, (lo0, hi0))
        t = jax.lax.bitcast_convert_type(lo, jnp.float32)

        c_gt = jnp.sum((v > t).astype(jnp.int32), axis=1, keepdims=True)
        m = K - c_gt
        eq = v == t
        col1 = jax.lax.broadcasted_iota(jnp.int32, (R, CP), 1) + 1

        def istep(_, lohi):
            lo, hi = lohi
            mid = (lo + hi) // 2
            cnt = jnp.sum((eq & (col1 <= mid)).astype(jnp.int32), axis=1,
                          keepdims=True)
            ge = cnt >= m
            return jnp.where(ge, lo, mid), jnp.where(ge, mid, hi)

        ilo0 = jnp.zeros((R, 1), jnp.int32)
        ihi0 = jnp.full((R, 1), CP, jnp.int32)
        _, ihi = jax.lax.fori_loop(0, 14, istep, (ilo0, ihi0))

        mask = (v > t) | (eq & (col1 <= ihi))
        out_ref[...] = (v * mask.astype(jnp.float32))[:, :N]


def _nodevecs(emb0, emb1, W0, b0, W1, b1):
    bs = 1000
    return pl.pallas_call(
        _nodevec_body,
        grid=(N // bs,),
        in_specs=[
            pl.BlockSpec((bs, D), lambda i: (i, 0)),
            pl.BlockSpec((bs, D), lambda i: (i, 0)),
            pl.BlockSpec((D, D), lambda i: (0, 0)),
            pl.BlockSpec((1, D), lambda i: (0, 0)),
            pl.BlockSpec((D, D), lambda i: (0, 0)),
            pl.BlockSpec((1, D), lambda i: (0, 0)),
        ],
        out_specs=[
            pl.BlockSpec((bs, D), lambda i: (i, 0)),
            pl.BlockSpec((bs, D), lambda i: (i, 0)),
        ],
        out_shape=[
            jax.ShapeDtypeStruct((N, D), jnp.float32),
            jax.ShapeDtypeStruct((N, D), jnp.float32),
        ],
    )(emb0, emb1, W0.T, b0.reshape(1, D), W1.T, b1.reshape(1, D))


def _masked_adj(H, GT, U, S):
    return pl.pallas_call(
        _adj_body,
        grid=(N // R,),
        in_specs=[
            pl.BlockSpec((R, 2 * D), lambda i: (i, 0)),
            pl.BlockSpec((2 * D, CP), lambda i: (0, 0)),
            pl.BlockSpec((CH, CH), lambda i: (0, 0)),
            pl.BlockSpec((NCH, NCH), lambda i: (0, 0)),
        ],
        out_specs=pl.BlockSpec((R, N), lambda i: (i, 0)),
        out_shape=jax.ShapeDtypeStruct((N, N), jnp.float32),
    )(H, GT, U, S)


def kernel(emb0, emb1, W0, b0, W1, b1, k):
    nv1, nv2 = _nodevecs(emb0, emb1, W0, b0, W1, b1)
    nv1b = nv1.astype(jnp.bfloat16)
    nv2b = nv2.astype(jnp.bfloat16)
    H = jnp.concatenate([nv1b, nv2b], axis=1)
    GT = jnp.concatenate([nv2b.T, -nv1b.T], axis=0)
    GT = jnp.pad(GT, ((0, 0), (0, CP - N)))
    U = jnp.triu(jnp.ones((CH, CH), jnp.bfloat16))
    S = jnp.triu(jnp.ones((NCH, NCH), jnp.bfloat16), k=1)
    return _masked_adj(H, GT, U, S)


# speedup vs baseline: 4.1059x; 2.2547x over previous
import jax
import jax.numpy as jnp
from jax.experimental import pallas as pl
from jax.experimental.pallas import tpu as pltpu

N = 10000
D = 128
CP = 10240
CH = 128
NCH = CP // CH
NFULL = N // CH
NREM = N - NFULL * CH
K = 32
ALPHA = 3.0
JF = 4
R = 200
ONE_BITS = 0x3F800000


def _nodevec_body(e0_ref, e1_ref, w0t_ref, b0_ref, w1t_ref, b1_ref,
                  nv1_ref, nv2_ref):
    a0 = jax.lax.dot_general(e0_ref[...].astype(jnp.bfloat16),
                             w0t_ref[...].astype(jnp.bfloat16),
                             (((1,), (0,)), ((), ())),
                             preferred_element_type=jnp.float32)
    a1 = jax.lax.dot_general(e1_ref[...].astype(jnp.bfloat16),
                             w1t_ref[...].astype(jnp.bfloat16),
                             (((1,), (0,)), ((), ())),
                             preferred_element_type=jnp.float32)
    nv1_ref[...] = jnp.tanh(ALPHA * (a0 + b0_ref[...]))
    nv2_ref[...] = jnp.tanh(ALPHA * (a1 + b1_ref[...]))


def _adj_body(h_ref, gt_ref, u_ref, s_ref, out_ref):
    h = h_ref[...]
    def xstep(_, lohi):
        lo, hi = lohi
        mid = lo + (hi - lo) // 2
        x = jax.lax.bitcast_convert_type(mid, jnp.float32)
        sat = jnp.tanh(ALPHA * x) >= 1.0
        return jnp.where(sat, lo, mid), jnp.where(sat, mid, hi)

    xlo0 = jnp.full((1, 1), 0x40000000, jnp.int32)
    xhi0 = jnp.full((1, 1), 0x41000000, jnp.int32)
    _, xhi = jax.lax.fori_loop(0, 26, xstep, (xlo0, xhi0))
    a_c = jax.lax.bitcast_convert_type(xhi, jnp.float32)

    u = u_ref[...]

    a_head = jax.lax.dot_general(h, gt_ref[:, :JF * CH],
                                 (((1,), (0,)), ((), ())),
                                 preferred_element_type=jnp.float32)
    ones_h = a_head >= a_c
    pres_h = []
    tots_h = []
    for j in range(JF):
        eqb = ones_h[:, j * CH:(j + 1) * CH].astype(jnp.bfloat16)
        pre = jax.lax.dot_general(eqb, u, (((1,), (0,)), ((), ())),
                                  preferred_element_type=jnp.float32)
        pres_h.append(pre)
        tots_h.append(pre[:, CH - 1:CH])
    tot_h = jnp.concatenate(tots_h, axis=1)
    offs_h = jax.lax.dot_general(tot_h.astype(jnp.bfloat16),
                                 s_ref[:JF, :JF], (((1,), (0,)), ((), ())),
                                 preferred_element_type=jnp.float32)
    c_head = offs_h[:, JF - 1:JF] + tot_h[:, JF - 1:JF]
    ffast = jnp.min(c_head) >= K

    @pl.when(ffast)
    def _():
        for j in range(JF):
            gp = pres_h[j] + offs_h[:, j:j + 1]
            keep = ones_h[:, j * CH:(j + 1) * CH] & (gp <= K)
            out_ref[:, j * CH:(j + 1) * CH] = keep.astype(jnp.float32)
        out_ref[:, JF * CH:N] = jnp.zeros((R, N - JF * CH), jnp.float32)

    @pl.when(jnp.logical_not(ffast))
    def _():
        a_tail = jax.lax.dot_general(h, gt_ref[:, JF * CH:],
                                     (((1,), (0,)), ((), ())),
                                     preferred_element_type=jnp.float32)
        a = jnp.concatenate([a_head, a_tail], axis=1)
        ones = a >= a_c

        pres = []
        tots = []
        for j in range(NCH):
            eqb = ones[:, j * CH:(j + 1) * CH].astype(jnp.bfloat16)
            pre = jax.lax.dot_general(eqb, u, (((1,), (0,)), ((), ())),
                                      preferred_element_type=jnp.float32)
            pres.append(pre)
            tots.append(pre[:, CH - 1:CH])
        tot = jnp.concatenate(tots, axis=1)
        offs = jax.lax.dot_general(tot.astype(jnp.bfloat16), s_ref[...],
                                   (((1,), (0,)), ((), ())),
                                   preferred_element_type=jnp.float32)
        n_tot = offs[:, NCH - 1:NCH] + tot[:, NCH - 1:NCH]
        fast = jnp.min(n_tot) >= K

        @pl.when(fast)
        def _():
            for j in range(NFULL + 1):
                gp = pres[j] + offs[:, j:j + 1]
                keep = ones[:, j * CH:(j + 1) * CH] & (gp <= K)
                outj = keep.astype(jnp.float32)
                if j < NFULL:
                    out_ref[:, j * CH:(j + 1) * CH] = outj
                else:
                    out_ref[:, j * CH:j * CH + NREM] = outj[:, :NREM]

        @pl.when(jnp.logical_not(fast))
        def _():
            _slow_exact(a, out_ref)


def _slow_exact(a, out_ref):
        v = jnp.maximum(jnp.tanh(ALPHA * a), 0.0)
        bits = jax.lax.bitcast_convert_type(v, jnp.int32)

        def step(_, lohi):
            lo, hi = lohi
            mid = lo + (hi - lo) // 2
            cnt = jnp.sum((bits >= mid).astype(jnp.int32), axis=1,
                          keepdims=True)
            ge = cnt >= K
            return jnp.where(ge, mid, lo), jnp.where(ge, hi, mid)

        lo0 = jnp.zeros((R, 1), jnp.int32)
        hi0 = jnp.full((R, 1), ONE_BITS + 1, jnp.int32)
        lo, _ = jax.lax.fori_loop(0, 31, step, (lo0, hi0))
        t = jax.lax.bitcast_convert_type(lo, jnp.float32)

        c_gt = jnp.sum((v > t).astype(jnp.int32), axis=1, keepdims=True)
        m = K - c_gt
        eq = v == t
        col1 = jax.lax.broadcasted_iota(jnp.int32, (R, CP), 1) + 1

        def istep(_, lohi):
            lo, hi = lohi
            mid = (lo + hi) // 2
            cnt = jnp.sum((eq & (col1 <= mid)).astype(jnp.int32), axis=1,
                          keepdims=True)
            ge = cnt >= m
            return jnp.where(ge, lo, mid), jnp.where(ge, mid, hi)

        ilo0 = jnp.zeros((R, 1), jnp.int32)
        ihi0 = jnp.full((R, 1), CP, jnp.int32)
        _, ihi = jax.lax.fori_loop(0, 14, istep, (ilo0, ihi0))

        mask = (v > t) | (eq & (col1 <= ihi))
        out_ref[...] = (v * mask.astype(jnp.float32))[:, :N]


def _nodevecs(emb0, emb1, W0, b0, W1, b1):
    bs = 1000
    return pl.pallas_call(
        _nodevec_body,
        grid=(N // bs,),
        in_specs=[
            pl.BlockSpec((bs, D), lambda i: (i, 0)),
            pl.BlockSpec((bs, D), lambda i: (i, 0)),
            pl.BlockSpec((D, D), lambda i: (0, 0)),
            pl.BlockSpec((1, D), lambda i: (0, 0)),
            pl.BlockSpec((D, D), lambda i: (0, 0)),
            pl.BlockSpec((1, D), lambda i: (0, 0)),
        ],
        out_specs=[
            pl.BlockSpec((bs, D), lambda i: (i, 0)),
            pl.BlockSpec((bs, D), lambda i: (i, 0)),
        ],
        out_shape=[
            jax.ShapeDtypeStruct((N, D), jnp.float32),
            jax.ShapeDtypeStruct((N, D), jnp.float32),
        ],
    )(emb0, emb1, W0.T, b0.reshape(1, D), W1.T, b1.reshape(1, D))


def _masked_adj(H, GT, U, S):
    return pl.pallas_call(
        _adj_body,
        grid=(N // R,),
        in_specs=[
            pl.BlockSpec((R, 2 * D), lambda i: (i, 0)),
            pl.BlockSpec((2 * D, CP), lambda i: (0, 0)),
            pl.BlockSpec((CH, CH), lambda i: (0, 0)),
            pl.BlockSpec((NCH, NCH), lambda i: (0, 0)),
        ],
        out_specs=pl.BlockSpec((R, N), lambda i: (i, 0)),
        out_shape=jax.ShapeDtypeStruct((N, N), jnp.float32),
    )(H, GT, U, S)


def kernel(emb0, emb1, W0, b0, W1, b1, k):
    nv1, nv2 = _nodevecs(emb0, emb1, W0, b0, W1, b1)
    nv1b = nv1.astype(jnp.bfloat16)
    nv2b = nv2.astype(jnp.bfloat16)
    H = jnp.concatenate([nv1b, nv2b], axis=1)
    GT = jnp.concatenate([nv2b.T, -nv1b.T], axis=0)
    GT = jnp.pad(GT, ((0, 0), (0, CP - N)))
    U = jnp.triu(jnp.ones((CH, CH), jnp.bfloat16))
    S = jnp.triu(jnp.ones((NCH, NCH), jnp.bfloat16), k=1)
    return _masked_adj(H, GT, U, S)


# speedup vs baseline: 4.1354x; 1.0072x over previous
import jax
import jax.numpy as jnp
from jax.experimental import pallas as pl

N = 10000
D = 128
CP = 10240
CH = 128
NCH = CP // CH
NFULL = N // CH
NREM = N - NFULL * CH
K = 32
ALPHA = 3.0
JF = 4
R = 200
ONE_BITS = 0x3F800000


def _nodevec_body(e0_ref, e1_ref, w0t_ref, b0_ref, w1t_ref, b1_ref,
                  nv1_ref, nv2_ref):
    a0 = jax.lax.dot_general(e0_ref[...].astype(jnp.bfloat16),
                             w0t_ref[...].astype(jnp.bfloat16),
                             (((1,), (0,)), ((), ())),
                             preferred_element_type=jnp.float32)
    a1 = jax.lax.dot_general(e1_ref[...].astype(jnp.bfloat16),
                             w1t_ref[...].astype(jnp.bfloat16),
                             (((1,), (0,)), ((), ())),
                             preferred_element_type=jnp.float32)
    nv1_ref[...] = jnp.tanh(ALPHA * (a0 + b0_ref[...]))
    nv2_ref[...] = jnp.tanh(ALPHA * (a1 + b1_ref[...]))


def _adj_body(h_ref, gt_ref, u_ref, s_ref, out_ref):
    h = h_ref[...]
    def xstep(_, lohi):
        lo, hi = lohi
        mid = lo + (hi - lo) // 2
        x = jax.lax.bitcast_convert_type(mid, jnp.float32)
        sat = jnp.tanh(ALPHA * x) >= 1.0
        return jnp.where(sat, lo, mid), jnp.where(sat, mid, hi)

    xlo0 = jnp.full((1, 1), 0x40000000, jnp.int32)
    xhi0 = jnp.full((1, 1), 0x41000000, jnp.int32)
    _, xhi = jax.lax.fori_loop(0, 26, xstep, (xlo0, xhi0))
    a_c = jax.lax.bitcast_convert_type(xhi, jnp.float32)

    u = u_ref[...]

    a_head = jax.lax.dot_general(h, gt_ref[:, :JF * CH],
                                 (((1,), (0,)), ((), ())),
                                 preferred_element_type=jnp.float32)
    ones_h = a_head >= a_c
    pres_h = []
    tots_h = []
    for j in range(JF):
        eqb = ones_h[:, j * CH:(j + 1) * CH].astype(jnp.bfloat16)
        pre = jax.lax.dot_general(eqb, u, (((1,), (0,)), ((), ())),
                                  preferred_element_type=jnp.float32)
        pres_h.append(pre)
        tots_h.append(pre[:, CH - 1:CH])
    tot_h = jnp.concatenate(tots_h, axis=1)
    offs_h = jax.lax.dot_general(tot_h.astype(jnp.bfloat16),
                                 s_ref[:JF, :JF], (((1,), (0,)), ((), ())),
                                 preferred_element_type=jnp.float32)
    c_head = offs_h[:, JF - 1:JF] + tot_h[:, JF - 1:JF]
    ffast = jnp.min(c_head) >= K

    @pl.when(ffast)
    def _():
        for j in range(JF):
            gp = pres_h[j] + offs_h[:, j:j + 1]
            keep = ones_h[:, j * CH:(j + 1) * CH] & (gp <= K)
            out_ref[:, j * CH:(j + 1) * CH] = keep.astype(jnp.float32)
        out_ref[:, JF * CH:N] = jnp.zeros((R, N - JF * CH), jnp.float32)

    @pl.when(jnp.logical_not(ffast))
    def _():
        a_tail = jax.lax.dot_general(h, gt_ref[:, JF * CH:],
                                     (((1,), (0,)), ((), ())),
                                     preferred_element_type=jnp.float32)
        a = jnp.concatenate([a_head, a_tail], axis=1)
        ones = a >= a_c

        pres = []
        tots = []
        for j in range(NCH):
            eqb = ones[:, j * CH:(j + 1) * CH].astype(jnp.bfloat16)
            pre = jax.lax.dot_general(eqb, u, (((1,), (0,)), ((), ())),
                                      preferred_element_type=jnp.float32)
            pres.append(pre)
            tots.append(pre[:, CH - 1:CH])
        tot = jnp.concatenate(tots, axis=1)
        offs = jax.lax.dot_general(tot.astype(jnp.bfloat16), s_ref[...],
                                   (((1,), (0,)), ((), ())),
                                   preferred_element_type=jnp.float32)
        n_tot = offs[:, NCH - 1:NCH] + tot[:, NCH - 1:NCH]
        fast = jnp.min(n_tot) >= K

        @pl.when(fast)
        def _():
            for j in range(NFULL + 1):
                gp = pres[j] + offs[:, j:j + 1]
                keep = ones[:, j * CH:(j + 1) * CH] & (gp <= K)
                outj = keep.astype(jnp.float32)
                if j < NFULL:
                    out_ref[:, j * CH:(j + 1) * CH] = outj
                else:
                    out_ref[:, j * CH:j * CH + NREM] = outj[:, :NREM]

        @pl.when(jnp.logical_not(fast))
        def _():
            _slow_exact(a, out_ref)


def _slow_exact(a, out_ref):
        v = jnp.maximum(jnp.tanh(ALPHA * a), 0.0)
        bits = jax.lax.bitcast_convert_type(v, jnp.int32)

        def step(_, lohi):
            lo, hi = lohi
            mid = lo + (hi - lo) // 2
            cnt = jnp.sum((bits >= mid).astype(jnp.int32), axis=1,
                          keepdims=True)
            ge = cnt >= K
            return jnp.where(ge, mid, lo), jnp.where(ge, hi, mid)

        lo0 = jnp.zeros((R, 1), jnp.int32)
        hi0 = jnp.full((R, 1), ONE_BITS + 1, jnp.int32)
        lo, _ = jax.lax.fori_loop(0, 31, step, (lo0, hi0))
        t = jax.lax.bitcast_convert_type(lo, jnp.float32)

        c_gt = jnp.sum((v > t).astype(jnp.int32), axis=1, keepdims=True)
        m = K - c_gt
        eq = v == t
        col1 = jax.lax.broadcasted_iota(jnp.int32, (R, CP), 1) + 1

        def istep(_, lohi):
            lo, hi = lohi
            mid = (lo + hi) // 2
            cnt = jnp.sum((eq & (col1 <= mid)).astype(jnp.int32), axis=1,
                          keepdims=True)
            ge = cnt >= m
            return jnp.where(ge, lo, mid), jnp.where(ge, mid, hi)

        ilo0 = jnp.zeros((R, 1), jnp.int32)
        ihi0 = jnp.full((R, 1), CP, jnp.int32)
        _, ihi = jax.lax.fori_loop(0, 14, istep, (ilo0, ihi0))

        mask = (v > t) | (eq & (col1 <= ihi))
        out_ref[...] = (v * mask.astype(jnp.float32))[:, :N]


def _nodevecs(emb0, emb1, W0, b0, W1, b1):
    bs = 1000
    return pl.pallas_call(
        _nodevec_body,
        grid=(N // bs,),
        in_specs=[
            pl.BlockSpec((bs, D), lambda i: (i, 0)),
            pl.BlockSpec((bs, D), lambda i: (i, 0)),
            pl.BlockSpec((D, D), lambda i: (0, 0)),
            pl.BlockSpec((1, D), lambda i: (0, 0)),
            pl.BlockSpec((D, D), lambda i: (0, 0)),
            pl.BlockSpec((1, D), lambda i: (0, 0)),
        ],
        out_specs=[
            pl.BlockSpec((bs, D), lambda i: (i, 0)),
            pl.BlockSpec((bs, D), lambda i: (i, 0)),
        ],
        out_shape=[
            jax.ShapeDtypeStruct((N, D), jnp.float32),
            jax.ShapeDtypeStruct((N, D), jnp.float32),
        ],
    )(emb0, emb1, W0.T, b0.reshape(1, D), W1.T, b1.reshape(1, D))


def _masked_adj(H, GT, U, S):
    return pl.pallas_call(
        _adj_body,
        grid=(N // R,),
        in_specs=[
            pl.BlockSpec((R, 2 * D), lambda i: (i, 0)),
            pl.BlockSpec((2 * D, CP), lambda i: (0, 0)),
            pl.BlockSpec((CH, CH), lambda i: (0, 0)),
            pl.BlockSpec((NCH, NCH), lambda i: (0, 0)),
        ],
        out_specs=pl.BlockSpec((R, N), lambda i: (i, 0)),
        out_shape=jax.ShapeDtypeStruct((N, N), jnp.float32),
    )(H, GT, U, S)


def kernel(emb0, emb1, W0, b0, W1, b1, k):
    nv1, nv2 = _nodevecs(emb0, emb1, W0, b0, W1, b1)
    nv1b = nv1.astype(jnp.bfloat16)
    nv2b = nv2.astype(jnp.bfloat16)
    H = jnp.concatenate([nv1b, nv2b], axis=1)
    GT = jnp.concatenate([nv2b.T, -nv1b.T], axis=0)
    GT = jnp.pad(GT, ((0, 0), (0, CP - N)))
    U = jnp.triu(jnp.ones((CH, CH), jnp.bfloat16))
    S = jnp.triu(jnp.ones((NCH, NCH), jnp.bfloat16), k=1)
    return _masked_adj(H, GT, U, S)
